# baseline (speedup 1.0000x reference)
import functools

import jax
import jax.numpy as jnp
from jax import lax
from jax.experimental import pallas as pl
from jax.experimental.pallas import tpu as pltpu
from jax.experimental.pallas import tpu_sc as plsc

_L = 2048
_NSUB = 32
_V = 16


def _expert_body(xt_ref, wg_ref, bg_ref, w1_ref, b1_ref, w2_ref, b2_ref,
                 w3_ref, b3_ref, lg_ref, eo_ref):
    f32 = jnp.float32
    bf16 = jnp.bfloat16
    xtb = xt_ref[...]
    lg_ref[...] = (jnp.dot(wg_ref[...], xtb, preferred_element_type=f32)
                   + bg_ref[...])
    h1 = jnp.maximum(
        jnp.dot(w1_ref[...], xtb.astype(bf16), preferred_element_type=f32)
        + b1_ref[...], 0.0)
    h1 = h1.astype(bf16)
    h2a = jnp.maximum(
        jnp.dot(w2_ref[0], h1[:256], preferred_element_type=f32)
        + b2_ref[:256], 0.0)
    h2b = jnp.maximum(
        jnp.dot(w2_ref[1], h1[256:], preferred_element_type=f32)
        + b2_ref[256:], 0.0)
    eo_ref[...] = (
        jnp.dot(w3_ref[:, :256], h2a.astype(bf16), preferred_element_type=f32)
        + jnp.dot(w3_ref[:, 256:], h2b.astype(bf16), preferred_element_type=f32)
        + b3_ref[...])


def _make_route_combine(n, e, c):
    mesh = plsc.VectorSubcoreMesh(core_axis_name="c", subcore_axis_name="s")

    @functools.partial(
        pl.kernel, mesh=mesh,
        out_type=jax.ShapeDtypeStruct((n,), jnp.float32),
        scratch_types=[
            pltpu.VMEM((e * c,), jnp.float32),
            pltpu.VMEM((e * c,), jnp.float32),
            pltpu.VMEM((c,), jnp.float32),
        ],
    )
    def route_combine(lg_hbm, eo_hbm, out_hbm, lbuf, eobuf, obuf):
        wid = lax.axis_index("s") * 2 + lax.axis_index("c")
        base = wid * c
        for ee in range(e):
            pltpu.sync_copy(lg_hbm.at[pl.ds(ee * n + base, c)],
                            lbuf.at[pl.ds(ee * c, c)])
            pltpu.sync_copy(eo_hbm.at[pl.ds(ee * n + base, c)],
                            eobuf.at[pl.ds(ee * c, c)])

        def group(jg, carry):
            j0 = jg * _V
            ls = [lbuf[pl.ds(ee * c + j0, _V)] for ee in range(e)]
            v1 = ls[0]
            i1 = jnp.zeros((_V,), jnp.float32)
            for ee in range(1, e):
                cc = ls[ee] > v1
                v1 = jnp.where(cc, ls[ee], v1)
                i1 = jnp.where(cc, float(ee), i1)
            v2 = jnp.full((_V,), -jnp.inf, jnp.float32)
            i2 = jnp.zeros((_V,), jnp.float32)
            for ee in range(e):
                cc = jnp.logical_and(ls[ee] > v2, i1 != float(ee))
                v2 = jnp.where(cc, ls[ee], v2)
                i2 = jnp.where(cc, float(ee), i2)
            g1 = 1.0 / (1.0 + jnp.exp(v2 - v1))
            g2 = 1.0 - g1
            zz = jnp.zeros((_V,), jnp.float32)
            acc = zz
            for ee in range(e):
                we = jnp.where(i1 == float(ee), g1,
                               jnp.where(i2 == float(ee), g2, zz))
                acc = acc + we * eobuf[pl.ds(ee * c + j0, _V)]
            obuf[pl.ds(j0, _V)] = acc
            return carry

        lax.fori_loop(0, c // _V, group, 0)
        pltpu.sync_copy(obuf, out_hbm.at[pl.ds(base, c)])

    return route_combine


@jax.jit
def kernel(x, Wg, bg, W1, b1, W2, b2, W3, b3):
    n, d = x.shape
    e, _, h = W1.shape
    g = 4
    ng = e // g
    bf16 = jnp.bfloat16
    c = n // _NSUB

    xt = x.T
    wgt = Wg.T
    bgc = bg.reshape(e, 1)
    w1t = W1.transpose(0, 2, 1).reshape(e * h, d).astype(bf16)
    b1c = b1.reshape(e * h, 1)
    eyeg = jnp.eye(g, dtype=W2.dtype)
    w2t = jnp.einsum('ij,gjhk->gikjh', eyeg,
                     W2.reshape(ng, g, h, h)).reshape(ng, g * h, g * h)
    w2t = w2t.astype(bf16)
    b2c = b2.reshape(e * h, 1)
    w3t = jnp.einsum('eh,ef->efh', W3[:, :, 0],
                     jnp.eye(e, dtype=W3.dtype)).reshape(e, e * h).astype(bf16)
    b3c = b3.reshape(e, 1)

    nstep = n // _L
    lg, eo = pl.pallas_call(
        _expert_body,
        grid=(nstep,),
        in_specs=[
            pl.BlockSpec((d, _L), lambda i: (0, i)),
            pl.BlockSpec((e, d), lambda i: (0, 0)),
            pl.BlockSpec((e, 1), lambda i: (0, 0)),
            pl.BlockSpec((e * h, d), lambda i: (0, 0)),
            pl.BlockSpec((e * h, 1), lambda i: (0, 0)),
            pl.BlockSpec((ng, g * h, g * h), lambda i: (0, 0, 0)),
            pl.BlockSpec((e * h, 1), lambda i: (0, 0)),
            pl.BlockSpec((e, e * h), lambda i: (0, 0)),
            pl.BlockSpec((e, 1), lambda i: (0, 0)),
        ],
        out_specs=[
            pl.BlockSpec((e, _L), lambda i: (0, i)),
            pl.BlockSpec((e, _L), lambda i: (0, i)),
        ],
        out_shape=[
            jax.ShapeDtypeStruct((e, n), jnp.float32),
            jax.ShapeDtypeStruct((e, n), jnp.float32),
        ],
        compiler_params=pltpu.CompilerParams(
            dimension_semantics=("arbitrary",)),
    )(xt, wgt, bgc, w1t, b1c, w2t, b2c, w3t, b3c)

    lf = lg.reshape(-1)
    eof = eo.reshape(-1)
    out = _make_route_combine(n, e, c)(lf, eof)
    return out.reshape(n, 1)
